# baseline (speedup 1.0000x reference)
import jax
import jax.numpy as jnp
from jax.experimental import pallas as pl

_NUM_USERS = 25000
_NUM_ITEMS = 25000
_N = _NUM_USERS + _NUM_ITEMS
_NUM_ENT = 100000
_N_LAYERS = 3


def _mean4_body(e0, e1, e2, e3, o):
    o[...] = (e0[...] + e1[...] + e2[...] + e3[...]) * 0.25


def kernel(embedding_user, embedding_item, embedding_entity, item_entities, edge_index, edge_values):
    entity_embs = embedding_entity[item_entities]
    padding_mask = (item_entities != _NUM_ENT).astype(jnp.float32)
    entity_embs = entity_embs * padding_mask[..., None]
    entity_embs_sum = entity_embs.sum(axis=1)
    denom = padding_mask.sum(axis=-1)[:, None]
    entity_embs_mean = jnp.nan_to_num(entity_embs_sum / denom)
    items_emb = embedding_item + entity_embs_mean

    all_emb = jnp.concatenate([embedding_user, items_emb], axis=0)
    embs = [all_emb]
    src = edge_index[0]
    dst = edge_index[1]
    for _ in range(_N_LAYERS):
        msgs = edge_values[:, None] * all_emb[src]
        all_emb = jax.ops.segment_sum(msgs, dst, num_segments=_N)
        embs.append(all_emb)

    blk = 1000
    light_out = pl.pallas_call(
        _mean4_body,
        out_shape=jax.ShapeDtypeStruct((_N, 64), jnp.float32),
        grid=(_N // blk,),
        in_specs=[pl.BlockSpec((blk, 64), lambda i: (i, 0))] * 4,
        out_specs=pl.BlockSpec((blk, 64), lambda i: (i, 0)),
    )(embs[0], embs[1], embs[2], embs[3])
    return light_out[:_NUM_USERS], light_out[_NUM_USERS:]


# speedup vs baseline: 4.0262x; 4.0262x over previous
import functools

import jax
import jax.numpy as jnp
from jax import lax
from jax.experimental import pallas as pl
from jax.experimental.pallas import tpu as pltpu
from jax.experimental.pallas import tpu_sc as plsc

NU = 25000
NI = 25000
NN = NU + NI
D = 64
K = 8
PAD = 100000
NL = 3
E = 800000
NC = 2
NS = 16

_MESH = plsc.VectorSubcoreMesh(core_axis_name="c", subcore_axis_name="s")

IG = 56
NG = 14
IW = IG * NG


def _kg_body(user_hbm, item_hbm, ent_hbm, ief_hbm, e0_hbm,
             idx_v, rows_v, item_v, out_v, ubuf_v):
    c = lax.axis_index("c")
    s = lax.axis_index("s")
    w = s * NC + c

    ub = pl.multiple_of(jnp.minimum(w * IW, NU - IW), 8)
    for t in range(2):
        pltpu.sync_copy(user_hbm.at[pl.ds(ub + t * 392, 392)], ubuf_v)
        pltpu.sync_copy(ubuf_v, e0_hbm.at[pl.ds(ub + t * 392, 392)])

    base = pl.multiple_of(jnp.minimum(w * IW, NI - IW), 8)

    def group(g, _):
        ib = pl.multiple_of(base + g * IG, 8)
        pltpu.sync_copy(ief_hbm.at[pl.ds(ib * K, IG * K)],
                        idx_v.at[pl.ds(0, IG * K)])
        pltpu.sync_copy(ent_hbm.at[idx_v.at[pl.ds(0, IG * K)]], rows_v)
        pltpu.sync_copy(item_hbm.at[pl.ds(ib, IG)], item_v)

        def item(i, _):
            acc = [jnp.zeros((16,), jnp.float32) for _ in range(4)]
            den = jnp.float32(0.0)
            nv = idx_v[pl.ds(i * K, 16)]
            mv = jnp.where(nv != PAD, jnp.float32(1.0), jnp.float32(0.0))
            for k in range(K):
                m = mv[k]
                den = den + m
                for q in range(4):
                    acc[q] = acc[q] + m * rows_v[i * K + k, pl.ds(q * 16, 16)]
            denv = jnp.full((16,), den, jnp.float32)
            invv = jnp.where(denv > 0.0, jnp.float32(1.0) / denv,
                             jnp.float32(0.0))
            for q in range(4):
                out_v[i, pl.ds(q * 16, 16)] = (
                    acc[q] * invv + item_v[i, pl.ds(q * 16, 16)])
            return 0

        lax.fori_loop(0, IG, item, 0)
        pltpu.sync_copy(out_v, e0_hbm.at[pl.ds(NU + ib, IG)])
        return 0

    lax.fori_loop(0, NG, group, 0)


_SC_PARAMS = pltpu.CompilerParams(use_tc_tiling_on_sc=False)

_kg_call = pl.kernel(
    _kg_body,
    out_type=jax.ShapeDtypeStruct((NN, D), jnp.float32),
    mesh=_MESH,
    compiler_params=_SC_PARAMS,
    scratch_types=[
        pltpu.VMEM((IG * K + 8,), jnp.int32),
        pltpu.VMEM((IG * K, D), jnp.float32),
        pltpu.VMEM((IG, D), jnp.float32),
        pltpu.VMEM((IG, D), jnp.float32),
        pltpu.VMEM((392, D), jnp.float32),
    ],
)

CH = 400
NCHUNK = E // NS // CH
DUM = 16
ACC_ROWS = NU + DUM
ZR = 56
SLICE = 1568


def _layer_body(emb_hbm, ei_hbm, ev_hbm, out_hbm,
                acc_sh, zbuf_v, src_v, dst_v, val_v, adj_v, rows_v):
    c = lax.axis_index("c")
    s = lax.axis_index("s")
    lo = c * NU

    def zrow(i, _):
        for q in range(4):
            zbuf_v[i, pl.ds(q * 16, 16)] = jnp.zeros((16,), jnp.float32)
        return 0

    lax.fori_loop(0, ZR, zrow, 0)
    zb = pl.multiple_of(jnp.minimum(s * SLICE, NU - SLICE), 8)

    def zcopy(t, _):
        pltpu.sync_copy(zbuf_v, acc_sh.at[pl.ds(zb + t * ZR, ZR)])
        return 0

    lax.fori_loop(0, SLICE // ZR, zcopy, 0)
    plsc.subcore_barrier()

    def chunk(j, _):
        ebase = pl.multiple_of(s * (E // NS) + j * CH, 8)
        pltpu.sync_copy(ei_hbm.at[0, pl.ds(ebase, CH)], src_v)
        pltpu.sync_copy(ei_hbm.at[1, pl.ds(ebase, CH)], dst_v)
        pltpu.sync_copy(ev_hbm.at[pl.ds(ebase, CH)], val_v)
        pltpu.sync_copy(emb_hbm.at[src_v], rows_v)

        def adj16(g, _):
            d = dst_v[pl.ds(g * 16, 16)]
            ld = d - lo
            inb = (ld >= 0) & (ld < NU)
            adj_v[pl.ds(g * 16, 16)] = jnp.where(inb, ld, NU + s)
            return 0

        lax.fori_loop(0, CH // 16, adj16, 0)

        def scale(g, _):
            vv = val_v[pl.ds(g * 16, 16)]
            for l in range(16):
                e = g * 16 + l
                v = vv[l]
                for q in range(4):
                    rows_v[e, pl.ds(q * 16, 16)] = (
                        rows_v[e, pl.ds(q * 16, 16)] * v)
            return 0

        lax.fori_loop(0, CH // 16, scale, 0)
        pltpu.sync_copy(rows_v, acc_sh.at[adj_v], add=True)
        return 0

    lax.fori_loop(0, NCHUNK, chunk, 0)
    plsc.subcore_barrier()
    wb = pl.multiple_of(jnp.minimum(s * SLICE, NU - SLICE), 8)
    pltpu.sync_copy(acc_sh.at[pl.ds(wb, SLICE)],
                    out_hbm.at[pl.ds(lo + wb, SLICE)])


_layer_call = pl.kernel(
    _layer_body,
    out_type=jax.ShapeDtypeStruct((NN, D), jnp.float32),
    mesh=_MESH,
    compiler_params=_SC_PARAMS,
    scratch_types=[
        pltpu.VMEM_SHARED((ACC_ROWS, D), jnp.float32),
        pltpu.VMEM((ZR, D), jnp.float32),
        pltpu.VMEM((CH,), jnp.int32),
        pltpu.VMEM((CH,), jnp.int32),
        pltpu.VMEM((CH,), jnp.float32),
        pltpu.VMEM((CH,), jnp.int32),
        pltpu.VMEM((CH, D), jnp.float32),
    ],
)

MR = 200


def _mean_body(e0_hbm, e1_hbm, e2_hbm, e3_hbm, out_hbm,
               b0, b1, b2, b3, ob):
    c = lax.axis_index("c")
    s = lax.axis_index("s")
    half = c * NU
    for t in range(8):
        base = pl.multiple_of(jnp.minimum((s * 8 + t) * MR, NU - MR), 8)
        pltpu.sync_copy(e0_hbm.at[pl.ds(half + base, MR)], b0)
        pltpu.sync_copy(e1_hbm.at[pl.ds(half + base, MR)], b1)
        pltpu.sync_copy(e2_hbm.at[pl.ds(half + base, MR)], b2)
        pltpu.sync_copy(e3_hbm.at[pl.ds(half + base, MR)], b3)

        def mrow(i, _):
            for q in range(4):
                dq = pl.ds(q * 16, 16)
                ob[i, dq] = (b0[i, dq] + b1[i, dq] + b2[i, dq]
                             + b3[i, dq]) * jnp.float32(0.25)
            return 0

        lax.fori_loop(0, MR, mrow, 0)

        pltpu.sync_copy(ob, out_hbm.at[pl.ds(half + base, MR)])


_mean_call = pl.kernel(
    _mean_body,
    out_type=jax.ShapeDtypeStruct((NN, D), jnp.float32),
    mesh=_MESH,
    compiler_params=_SC_PARAMS,
    scratch_types=[pltpu.VMEM((MR, D), jnp.float32) for _ in range(5)],
)


def kernel(embedding_user, embedding_item, embedding_entity, item_entities,
           edge_index, edge_values):
    ief = item_entities.reshape(-1).astype(jnp.int32)
    ei = edge_index.astype(jnp.int32)
    e0 = _kg_call(embedding_user, embedding_item, embedding_entity, ief)
    e1 = _layer_call(e0, ei, edge_values)
    e2 = _layer_call(e1, ei, edge_values)
    e3 = _layer_call(e2, ei, edge_values)
    light = _mean_call(e0, e1, e2, e3)
    return light[:NU], light[NU:]


# speedup vs baseline: 6.8556x; 1.7027x over previous
import functools

import jax
import jax.numpy as jnp
from jax import lax
from jax.experimental import pallas as pl
from jax.experimental.pallas import tpu as pltpu
from jax.experimental.pallas import tpu_sc as plsc

NU = 25000
NI = 25000
NN = NU + NI
D = 64
K = 8
PAD = 100000
NL = 3
E = 800000
NC = 2
NS = 16

_MESH = plsc.VectorSubcoreMesh(core_axis_name="c", subcore_axis_name="s")

IG = 56
NG = 14
IW = IG * NG


def _kg_body(user_hbm, item_hbm, ent_hbm, ief_hbm, e0_hbm,
             idx_v, rows_v, item_v, out_v, ubuf_v):
    c = lax.axis_index("c")
    s = lax.axis_index("s")
    w = s * NC + c

    ub = pl.multiple_of(jnp.minimum(w * IW, NU - IW), 8)
    for t in range(2):
        pltpu.sync_copy(user_hbm.at[pl.ds(ub + t * 392, 392)], ubuf_v)
        pltpu.sync_copy(ubuf_v, e0_hbm.at[pl.ds(ub + t * 392, 392)])

    base = pl.multiple_of(jnp.minimum(w * IW, NI - IW), 8)

    def group(g, _):
        ib = pl.multiple_of(base + g * IG, 8)
        pltpu.sync_copy(ief_hbm.at[pl.ds(ib * K, IG * K)],
                        idx_v.at[pl.ds(0, IG * K)])
        pltpu.sync_copy(ent_hbm.at[idx_v.at[pl.ds(0, IG * K)]], rows_v)
        pltpu.sync_copy(item_hbm.at[pl.ds(ib, IG)], item_v)

        def item(i, _):
            acc = [jnp.zeros((16,), jnp.float32) for _ in range(4)]
            den = jnp.float32(0.0)
            nv = idx_v[pl.ds(i * K, 16)]
            mv = jnp.where(nv != PAD, jnp.float32(1.0), jnp.float32(0.0))
            for k in range(K):
                m = mv[k]
                den = den + m
                for q in range(4):
                    acc[q] = acc[q] + m * rows_v[i * K + k, pl.ds(q * 16, 16)]
            denv = jnp.full((16,), den, jnp.float32)
            invv = jnp.where(denv > 0.0, jnp.float32(1.0) / denv,
                             jnp.float32(0.0))
            for q in range(4):
                out_v[i, pl.ds(q * 16, 16)] = (
                    acc[q] * invv + item_v[i, pl.ds(q * 16, 16)])
            return 0

        lax.fori_loop(0, IG, item, 0)
        pltpu.sync_copy(out_v, e0_hbm.at[pl.ds(NU + ib, IG)])
        return 0

    lax.fori_loop(0, NG, group, 0)


_SC_PARAMS = pltpu.CompilerParams(use_tc_tiling_on_sc=False)

_kg_call = pl.kernel(
    _kg_body,
    out_type=jax.ShapeDtypeStruct((NN, D), jnp.float32),
    mesh=_MESH,
    compiler_params=_SC_PARAMS,
    scratch_types=[
        pltpu.VMEM((IG * K + 8,), jnp.int32),
        pltpu.VMEM((IG * K, D), jnp.float32),
        pltpu.VMEM((IG, D), jnp.float32),
        pltpu.VMEM((IG, D), jnp.float32),
        pltpu.VMEM((392, D), jnp.float32),
    ],
)

CH = 200
PER = E // NS
NCHUNK = PER // CH
SLICE = 1568


def _layer_body(emb_hbm, ei_hbm, ev_hbm, zeros_hbm, out_hbm,
                acc_sh, src0, src1, dst0, dst1, val0, val1, adj0, adj1,
                rows0, rows1, isem0, isem1, gsem0, gsem1):
    c = lax.axis_index("c")
    s = lax.axis_index("s")
    lo = c * NU
    sink = s * 1563
    srcb = (src0, src1)
    dstb = (dst0, dst1)
    valb = (val0, val1)
    adjb = (adj0, adj1)
    rowsb = (rows0, rows1)
    isem = (isem0, isem1)
    gsem = (gsem0, gsem1)

    def idx_start(jj, b):
        jc = jnp.minimum(jj, NCHUNK - 1)
        ebase = pl.multiple_of(s * PER + jc * CH, 8)
        pltpu.async_copy(ei_hbm.at[0, pl.ds(ebase, CH)], srcb[b], isem[b])
        pltpu.async_copy(ei_hbm.at[1, pl.ds(ebase, CH)], dstb[b], isem[b])
        pltpu.async_copy(ev_hbm.at[pl.ds(ebase, CH)], valb[b], isem[b])

    def idx_wait(b):
        pltpu.make_async_copy(ei_hbm.at[0, pl.ds(0, CH)], srcb[b],
                              isem[b]).wait()
        pltpu.make_async_copy(ei_hbm.at[1, pl.ds(0, CH)], dstb[b],
                              isem[b]).wait()
        pltpu.make_async_copy(ev_hbm.at[pl.ds(0, CH)], valb[b],
                              isem[b]).wait()

    def gather_start(b):
        pltpu.async_copy(emb_hbm.at[srcb[b]], rowsb[b], gsem[b])

    def gather_wait(b):
        pltpu.make_async_copy(emb_hbm.at[pl.ds(0, CH)], rowsb[b],
                              gsem[b]).wait()

    def compute(b):
        def adj_group(off, lanes):
            d = dstb[b][pl.ds(off, 16)]
            ld = d - lo
            inb = (ld >= 0) & (ld < NU)
            adjb[b][pl.ds(off, 16)] = jnp.where(inb, ld, sink)
            mf = jnp.where(inb, jnp.float32(1.0), jnp.float32(0.0))
            valb[b][pl.ds(off, 16)] = valb[b][pl.ds(off, 16)] * mf

        def scale_group(off, lanes):
            vv = valb[b][pl.ds(off, 16)]
            for l in lanes:
                v = vv[l]
                e = off + l
                for q in range(4):
                    rowsb[b][e, pl.ds(q * 16, 16)] = (
                        rowsb[b][e, pl.ds(q * 16, 16)] * v)

        def grp(g, _):
            adj_group(g * 16, range(16))
            return 0

        lax.fori_loop(0, CH // 16, grp, 0)
        adj_group(CH - 16, range(16))

        def sgrp(g, _):
            scale_group(g * 16, range(16))
            return 0

        lax.fori_loop(0, CH // 16, sgrp, 0)
        scale_group(CH - 16, range(16 - (CH - CH // 16 * 16), 16))

    def section(j, b):
        idx_wait(1 - b)
        gather_start(1 - b)
        gather_wait(b)
        compute(b)
        idx_start(j + 2, b)
        pltpu.sync_copy(rowsb[b], acc_sh.at[adjb[b]], add=True)

    zb = pl.multiple_of(jnp.minimum(s * SLICE, NU - SLICE), 8)
    idx_start(0, 0)
    idx_start(1, 1)
    pltpu.sync_copy(zeros_hbm, acc_sh.at[pl.ds(zb, SLICE)])
    idx_wait(0)
    gather_start(0)
    plsc.subcore_barrier()

    def pair(g, _):
        section(2 * g, 0)
        section(2 * g + 1, 1)
        return 0

    lax.fori_loop(0, NCHUNK // 2, pair, 0)
    gather_wait(0)
    idx_wait(1)
    plsc.subcore_barrier()
    wb = pl.multiple_of(jnp.minimum(s * SLICE, NU - SLICE), 8)
    pltpu.sync_copy(acc_sh.at[pl.ds(wb, SLICE)],
                    out_hbm.at[pl.ds(lo + wb, SLICE)])


_layer_call = pl.kernel(
    _layer_body,
    out_type=jax.ShapeDtypeStruct((NN, D), jnp.float32),
    mesh=_MESH,
    compiler_params=_SC_PARAMS,
    scratch_types=[
        pltpu.VMEM_SHARED((NU, D), jnp.float32),
        pltpu.VMEM((CH,), jnp.int32),
        pltpu.VMEM((CH,), jnp.int32),
        pltpu.VMEM((CH,), jnp.int32),
        pltpu.VMEM((CH,), jnp.int32),
        pltpu.VMEM((CH,), jnp.float32),
        pltpu.VMEM((CH,), jnp.float32),
        pltpu.VMEM((CH,), jnp.int32),
        pltpu.VMEM((CH,), jnp.int32),
        pltpu.VMEM((CH, D), jnp.float32),
        pltpu.VMEM((CH, D), jnp.float32),
        pltpu.SemaphoreType.DMA,
        pltpu.SemaphoreType.DMA,
        pltpu.SemaphoreType.DMA,
        pltpu.SemaphoreType.DMA,
    ],
)

MR = 200


def _mean_body(e0_hbm, e1_hbm, e2_hbm, e3_hbm, out_hbm,
               b0, b1, b2, b3, ob):
    c = lax.axis_index("c")
    s = lax.axis_index("s")
    half = c * NU
    for t in range(8):
        base = pl.multiple_of(jnp.minimum((s * 8 + t) * MR, NU - MR), 8)
        pltpu.sync_copy(e0_hbm.at[pl.ds(half + base, MR)], b0)
        pltpu.sync_copy(e1_hbm.at[pl.ds(half + base, MR)], b1)
        pltpu.sync_copy(e2_hbm.at[pl.ds(half + base, MR)], b2)
        pltpu.sync_copy(e3_hbm.at[pl.ds(half + base, MR)], b3)

        def mrow(i, _):
            for q in range(4):
                dq = pl.ds(q * 16, 16)
                ob[i, dq] = (b0[i, dq] + b1[i, dq] + b2[i, dq]
                             + b3[i, dq]) * jnp.float32(0.25)
            return 0

        lax.fori_loop(0, MR, mrow, 0)

        pltpu.sync_copy(ob, out_hbm.at[pl.ds(half + base, MR)])


_mean_call = pl.kernel(
    _mean_body,
    out_type=jax.ShapeDtypeStruct((NN, D), jnp.float32),
    mesh=_MESH,
    compiler_params=_SC_PARAMS,
    scratch_types=[pltpu.VMEM((MR, D), jnp.float32) for _ in range(5)],
)


def kernel(embedding_user, embedding_item, embedding_entity, item_entities,
           edge_index, edge_values):
    ief = item_entities.reshape(-1).astype(jnp.int32)
    ei = edge_index.astype(jnp.int32)
    zeros = jnp.zeros((SLICE, D), jnp.float32)
    e0 = _kg_call(embedding_user, embedding_item, embedding_entity, ief)
    e1 = _layer_call(e0, ei, edge_values, zeros)
    e2 = _layer_call(e1, ei, edge_values, zeros)
    e3 = _layer_call(e2, ei, edge_values, zeros)
    light = _mean_call(e0, e1, e2, e3)
    return light[:NU], light[NU:]
